# baseline (speedup 1.0000x reference)
import functools

import jax
import jax.numpy as jnp
from jax import lax
from jax.experimental import pallas as pl
from jax.experimental.pallas import tpu as pltpu
from jax.experimental.pallas import tpu_sc as plsc

N = 10000
E = 320000
NODE_IN = 128
NGRAPH = 64


BE = 2560


def _edge_mlp_body(er, We1, be1, We2, be2, Wl1, bl1, Wl2, bl2, t1, t2a, t2b):
    a = jnp.maximum(
        jnp.dot(er[...], We1[...], preferred_element_type=jnp.float32, precision=lax.Precision.HIGHEST) + be1[...], 0.0)
    ea = jnp.dot(a, We2[...], preferred_element_type=jnp.float32, precision=lax.Precision.HIGHEST) + be2[...]
    t1[...] = jnp.dot(ea, Wl1[...], preferred_element_type=jnp.float32, precision=lax.Precision.HIGHEST) + bl1[...]
    t2 = jnp.dot(ea, Wl2[...], preferred_element_type=jnp.float32, precision=lax.Precision.HIGHEST) + bl2[...]
    t2a[...] = t2[:, :128]
    t2b[...] = t2[:, 128:]


def _edge_mlp(edge_attr, We1, be1, We2, be2, Wl1, bl1, Wl2, bl2):
    full = lambda s: pl.BlockSpec(s, lambda i: (0,) * len(s))
    return pl.pallas_call(
        _edge_mlp_body,
        grid=(E // BE,),
        in_specs=[
            pl.BlockSpec((BE, 16), lambda i: (i, 0)),
            full((16, 256)), full((1, 256)),
            full((256, 256)), full((1, 256)),
            full((256, 128)), full((1, 128)),
            full((256, 256)), full((1, 256)),
        ],
        out_specs=[
            pl.BlockSpec((BE, 128), lambda i: (i, 0)),
            pl.BlockSpec((BE, 128), lambda i: (i, 0)),
            pl.BlockSpec((BE, 128), lambda i: (i, 0)),
        ],
        out_shape=[
            jax.ShapeDtypeStruct((E, 128), jnp.float32),
            jax.ShapeDtypeStruct((E, 128), jnp.float32),
            jax.ShapeDtypeStruct((E, 128), jnp.float32),
        ],
    )(edge_attr, We1, be1, We2, be2, Wl1, bl1, Wl2, bl2)


NTILES = 32
PW = E // NTILES
K = 8
NCH = PW // K
N_PAD = 10240
RPT = N_PAD // 16


def _sc_body(xh, th, src_h, dst_h, zero_h, part, idx_s, idx_d, rows, trow,
             acc, gsem):
    cid = lax.axis_index("c")
    sid = lax.axis_index("s")
    pltpu.sync_copy(zero_h.at[pl.ds(sid * RPT, RPT)],
                    acc.at[pl.ds(sid * RPT, RPT)])
    plsc.subcore_barrier()

    wid = sid * 2 + cid
    e0 = wid * PW

    def chunk(i, carry):
        base = e0 + i * K
        pltpu.sync_copy(src_h.at[pl.ds(base, K)], idx_s)
        pltpu.sync_copy(dst_h.at[pl.ds(base, K)], idx_d)
        gather = pltpu.async_copy(xh.at[idx_s], rows, gsem)
        pltpu.sync_copy(th.at[pl.ds(base, K)], trow)
        gather.wait()

        def rowloop(r, c2):
            for c in range(8):
                v = rows[r, pl.ds(c * 16, 16)] + trow[r, pl.ds(c * 16, 16)]
                rows[r, pl.ds(c * 16, 16)] = jnp.maximum(v, 0.0)
            return c2
        lax.fori_loop(0, K, rowloop, 0)
        pltpu.sync_copy(rows, acc.at[idx_d], add=True)
        return carry

    lax.fori_loop(0, NCH, chunk, 0)
    plsc.subcore_barrier()
    pltpu.sync_copy(acc.at[pl.ds(sid * RPT, RPT)],
                    part.at[cid, pl.ds(sid * RPT, RPT)])


def _sc_pass(x_slab, t_slab, src, dst, zeros):
    mesh = plsc.VectorSubcoreMesh(core_axis_name="c", subcore_axis_name="s")
    f = functools.partial(
        pl.kernel,
        out_type=jax.ShapeDtypeStruct((2, N_PAD, 128), jnp.float32),
        mesh=mesh,
        scratch_types=[
            pltpu.VMEM((K,), jnp.int32),
            pltpu.VMEM((K,), jnp.int32),
            pltpu.VMEM((K, 128), jnp.float32),
            pltpu.VMEM((K, 128), jnp.float32),
            pltpu.VMEM_SHARED((N_PAD, 128), jnp.float32),
            pltpu.SemaphoreType.DMA,
        ],
    )(_sc_body)
    return f(x_slab, t_slab, src, dst, zeros)


BN = 2000


def _node1_body(x, p1, A1, a1, A2, a2, x1a, x1b):
    h = x[...] + p1[0] + p1[1]
    h = jnp.maximum(jnp.dot(h, A1[...], preferred_element_type=jnp.float32, precision=lax.Precision.HIGHEST) + a1[...], 0.0)
    h = jnp.dot(h, A2[...], preferred_element_type=jnp.float32, precision=lax.Precision.HIGHEST) + a2[...]
    x1 = jnp.maximum(h, 0.0)
    x1a[...] = x1[:, :128]
    x1b[...] = x1[:, 128:]


def _node1(x, p1, A1, a1, A2, a2):
    full = lambda s: pl.BlockSpec(s, lambda i: (0,) * len(s))
    return pl.pallas_call(
        _node1_body,
        grid=(N // BN,),
        in_specs=[
            pl.BlockSpec((BN, 128), lambda i: (i, 0)),
            pl.BlockSpec((2, BN, 128), lambda i: (0, i, 0)),
            full((128, 256)), full((1, 256)),
            full((256, 256)), full((1, 256)),
        ],
        out_specs=[
            pl.BlockSpec((BN, 128), lambda i: (i, 0)),
            pl.BlockSpec((BN, 128), lambda i: (i, 0)),
        ],
        out_shape=[
            jax.ShapeDtypeStruct((N, 128), jnp.float32),
            jax.ShapeDtypeStruct((N, 128), jnp.float32),
        ],
    )(x, p1, A1, a1, A2, a2)


def _final_body(x1a, x1b, p2a, p2b, batch, B1, b1, B2, b2,
                L1, l1, L2, l2, H, hb, out, sums, cnt):
    i = pl.program_id(0)
    nsteps = pl.num_programs(0)

    @pl.when(i == 0)
    def _init():
        sums[...] = jnp.zeros_like(sums)
        cnt[...] = jnp.zeros_like(cnt)

    h = jnp.concatenate([x1a[...] + p2a[0] + p2a[1],
                         x1b[...] + p2b[0] + p2b[1]], axis=1)
    h = jnp.maximum(jnp.dot(h, B1[...], preferred_element_type=jnp.float32, precision=lax.Precision.HIGHEST) + b1[...], 0.0)
    h = jnp.dot(h, B2[...], preferred_element_type=jnp.float32, precision=lax.Precision.HIGHEST) + b2[...]
    x2 = jnp.maximum(h, 0.0)

    gid = lax.broadcasted_iota(jnp.int32, (BN, NGRAPH), 1)
    oh = jnp.where(batch[...] == gid, 1.0, 0.0).astype(jnp.float32)
    sums[...] += lax.dot_general(oh, x2, (((0,), (0,)), ((), ())),
                                 preferred_element_type=jnp.float32, precision=lax.Precision.HIGHEST)
    cnt[...] += lax.dot_general(oh, jnp.ones((BN, 1), jnp.float32),
                                (((0,), (0,)), ((), ())),
                                preferred_element_type=jnp.float32, precision=lax.Precision.HIGHEST)

    @pl.when(i == nsteps - 1)
    def _finish():
        g = sums[...] / jnp.maximum(cnt[...], 1.0)
        g = jnp.maximum(jnp.dot(g, L1[...], preferred_element_type=jnp.float32, precision=lax.Precision.HIGHEST) + l1[...], 0.0)
        g = jnp.maximum(jnp.dot(g, L2[...], preferred_element_type=jnp.float32, precision=lax.Precision.HIGHEST) + l2[...], 0.0)
        out[...] = jnp.dot(g, H[...], preferred_element_type=jnp.float32, precision=lax.Precision.HIGHEST) + hb[...]


def _final(x1a, x1b, p2a, p2b, batch2d, B1, b1, B2, b2, L1, l1, L2, l2, H, hb):
    full = lambda s: pl.BlockSpec(s, lambda i: (0,) * len(s))
    return pl.pallas_call(
        _final_body,
        grid=(N // BN,),
        in_specs=[
            pl.BlockSpec((BN, 128), lambda i: (i, 0)),
            pl.BlockSpec((BN, 128), lambda i: (i, 0)),
            pl.BlockSpec((2, BN, 128), lambda i: (0, i, 0)),
            pl.BlockSpec((2, BN, 128), lambda i: (0, i, 0)),
            pl.BlockSpec((BN, 1), lambda i: (i, 0)),
            full((256, 256)), full((1, 256)),
            full((256, 256)), full((1, 256)),
            full((256, 128)), full((1, 128)),
            full((128, 64)), full((1, 64)),
            full((64, 3)), full((1, 3)),
        ],
        out_specs=pl.BlockSpec((NGRAPH, 3), lambda i: (0, 0)),
        out_shape=jax.ShapeDtypeStruct((NGRAPH, 3), jnp.float32),
        scratch_shapes=[
            pltpu.VMEM((NGRAPH, 256), jnp.float32),
            pltpu.VMEM((NGRAPH, 1), jnp.float32),
        ],
    )(x1a, x1b, p2a, p2b, batch2d, B1, b1, B2, b2, L1, l1, L2, l2, H, hb)


def kernel(x, edge_index, edge_attr, batch, We1, be1, We2, be2, Wl1, bl1,
           A1, a1, A2, a2, Wl2, bl2, B1, b1, B2, b2, L1, l1, L2, l2,
           Hs, hs, Hp, hp, Hn, hn):
    src = edge_index[0]
    dst = edge_index[1]
    r = lambda b: b.reshape(1, -1)

    t1, t2a, t2b = _edge_mlp(edge_attr, We1, r(be1), We2, r(be2),
                             Wl1, r(bl1), Wl2, r(bl2))

    zeros = jnp.zeros((N_PAD, 128), jnp.float32)
    p1 = _sc_pass(x, t1, src, dst, zeros)
    x1a, x1b = _node1(x, p1, A1, r(a1), A2, r(a2))
    p2a = _sc_pass(x1a, t2a, src, dst, zeros)
    p2b = _sc_pass(x1b, t2b, src, dst, zeros)

    H = jnp.concatenate([Hs, Hp, Hn], axis=1)
    hb = jnp.stack([hs[0], hp[0], hn[0]]).reshape(1, 3)
    out = _final(x1a, x1b, p2a, p2b, batch.reshape(N, 1).astype(jnp.int32),
                 B1, r(b1), B2, r(b2), L1, r(l1), L2, r(l2), H, hb)
    return out[:, 0], out[:, 1], out[:, 2]


# speedup vs baseline: 1.6054x; 1.6054x over previous
import functools

import jax
import jax.numpy as jnp
from jax import lax
from jax.experimental import pallas as pl
from jax.experimental.pallas import tpu as pltpu
from jax.experimental.pallas import tpu_sc as plsc

N = 10000
E = 320000
NODE_IN = 128
NGRAPH = 64


BE = 2560


def _edge_mlp_body(er, We1, be1, We2, be2, Wl1, bl1, Wl2, bl2, t1, t2a, t2b):
    a = jnp.maximum(
        jnp.dot(er[...], We1[...], preferred_element_type=jnp.float32, precision=lax.Precision.HIGHEST) + be1[...], 0.0)
    ea = jnp.dot(a, We2[...], preferred_element_type=jnp.float32, precision=lax.Precision.HIGHEST) + be2[...]
    t1[...] = jnp.dot(ea, Wl1[...], preferred_element_type=jnp.float32, precision=lax.Precision.HIGHEST) + bl1[...]
    t2 = jnp.dot(ea, Wl2[...], preferred_element_type=jnp.float32, precision=lax.Precision.HIGHEST) + bl2[...]
    t2a[...] = t2[:, :128]
    t2b[...] = t2[:, 128:]


def _edge_mlp(edge_attr, We1, be1, We2, be2, Wl1, bl1, Wl2, bl2):
    full = lambda s: pl.BlockSpec(s, lambda i: (0,) * len(s))
    return pl.pallas_call(
        _edge_mlp_body,
        grid=(E // BE,),
        in_specs=[
            pl.BlockSpec((BE, 16), lambda i: (i, 0)),
            full((16, 256)), full((1, 256)),
            full((256, 256)), full((1, 256)),
            full((256, 128)), full((1, 128)),
            full((256, 256)), full((1, 256)),
        ],
        out_specs=[
            pl.BlockSpec((BE, 128), lambda i: (i, 0)),
            pl.BlockSpec((BE, 128), lambda i: (i, 0)),
            pl.BlockSpec((BE, 128), lambda i: (i, 0)),
        ],
        out_shape=[
            jax.ShapeDtypeStruct((E, 128), jnp.float32),
            jax.ShapeDtypeStruct((E, 128), jnp.float32),
            jax.ShapeDtypeStruct((E, 128), jnp.float32),
        ],
    )(edge_attr, We1, be1, We2, be2, Wl1, bl1, Wl2, bl2)


NTILES = 32
PW = E // NTILES
K = 80
G = K // 16
NCH = PW // K
N_PAD = 10240
RPT = N_PAD // 16
TRASH = N_PAD - K


def _sc_body(xh, th, src_h, dst_h, zero_h, part, idxb, dstb, rows, trow, sidx,
             ridx, occ80, acc, gsem, tsem):
    cid = lax.axis_index("c")
    sid = lax.axis_index("s")
    pltpu.sync_copy(zero_h.at[pl.ds(sid * RPT, RPT)],
                    acc.at[pl.ds(sid * RPT, RPT)])
    plsc.subcore_barrier()

    wid = sid * 2 + cid
    e0 = wid * PW
    lane = lax.iota(jnp.int32, 16)
    one = jnp.full((16,), 1, jnp.int32)

    def chunk(i, carry):
        b = lax.rem(i, 2)
        base = e0 + i * K
        pltpu.sync_copy(src_h.at[pl.ds(base, K)], idxb.at[0])
        pltpu.sync_copy(dst_h.at[pl.ds(base, K)], dstb.at[b])
        pltpu.async_copy(th.at[pl.ds(base, K)], trow, tsem).wait()
        pltpu.async_copy(xh.at[idxb.at[0]], rows.at[b], gsem).wait()

        for g in range(G):
            def rowloop(r, c2):
                for c in range(8):
                    v = rows[b, r, pl.ds(c * 16, 16)] + trow[r, pl.ds(c * 16, 16)]
                    rows[b, r, pl.ds(c * 16, 16)] = jnp.maximum(v, 0.0)
                return c2
            lax.fori_loop(g * 16, (g + 1) * 16, rowloop, 0)

        pltpu.sync_copy(rows.at[b], acc.at[dstb.at[b]], add=True)
        return carry

    lax.fori_loop(0, NCH, chunk, 0)
    plsc.subcore_barrier()
    pltpu.sync_copy(acc.at[pl.ds(sid * RPT, RPT)],
                    part.at[cid, pl.ds(sid * RPT, RPT)])


def _sc_pass(x_slab, t_slab, src, dst, zeros):
    mesh = plsc.VectorSubcoreMesh(core_axis_name="c", subcore_axis_name="s")
    f = functools.partial(
        pl.kernel,
        out_type=jax.ShapeDtypeStruct((2, N_PAD, 128), jnp.float32),
        mesh=mesh,
        compiler_params=pltpu.CompilerParams(needs_layout_passes=False),
        scratch_types=[
            pltpu.VMEM((2, K), jnp.int32),
            pltpu.VMEM((2, K), jnp.int32),
            pltpu.VMEM((2, K, 128), jnp.float32),
            pltpu.VMEM((K, 128), jnp.float32),
            pltpu.VMEM((K,), jnp.int32),
            pltpu.VMEM((K,), jnp.int32),
            pltpu.VMEM((K,), jnp.int32),
            pltpu.VMEM_SHARED((N_PAD, 128), jnp.float32),
            pltpu.SemaphoreType.DMA,
            pltpu.SemaphoreType.DMA,
        ],
    )(_sc_body)
    return f(x_slab, t_slab, src, dst, zeros)


def _scs_body(m_h, dst_h, zero_h, part, dstb, rows, acc, tsem):
    cid = lax.axis_index("c")
    sid = lax.axis_index("s")
    pltpu.sync_copy(zero_h.at[pl.ds(sid * RPT, RPT)],
                    acc.at[pl.ds(sid * RPT, RPT)])
    plsc.subcore_barrier()
    wid = sid * 2 + cid
    e0 = wid * PW

    def chunk(i, carry):
        base = e0 + i * K
        pltpu.sync_copy(dst_h.at[pl.ds(base, K)], dstb)
        pltpu.async_copy(m_h.at[pl.ds(base, K)], rows, tsem).wait()
        pltpu.sync_copy(rows, acc.at[dstb], add=True)
        return carry

    lax.fori_loop(0, NCH, chunk, 0)
    plsc.subcore_barrier()
    pltpu.sync_copy(acc.at[pl.ds(sid * RPT, RPT)],
                    part.at[cid, pl.ds(sid * RPT, RPT)])


def _sc_scatter(m, dst, zeros):
    mesh = plsc.VectorSubcoreMesh(core_axis_name="c", subcore_axis_name="s")
    f = functools.partial(
        pl.kernel,
        out_type=jax.ShapeDtypeStruct((2, N_PAD, 128), jnp.float32),
        mesh=mesh,
        compiler_params=pltpu.CompilerParams(needs_layout_passes=False),
        scratch_types=[
            pltpu.VMEM((K,), jnp.int32),
            pltpu.VMEM((K, 128), jnp.float32),
            pltpu.VMEM_SHARED((N_PAD, 128), jnp.float32),
            pltpu.SemaphoreType.DMA,
        ],
    )(_scs_body)
    return f(m, dst, zeros)


BN = 2000


def _node1_body(x, p1, A1, a1, A2, a2, x1a, x1b):
    h = x[...] + p1[0] + p1[1]
    h = jnp.maximum(jnp.dot(h, A1[...], preferred_element_type=jnp.float32, precision=lax.Precision.HIGHEST) + a1[...], 0.0)
    h = jnp.dot(h, A2[...], preferred_element_type=jnp.float32, precision=lax.Precision.HIGHEST) + a2[...]
    x1 = jnp.maximum(h, 0.0)
    x1a[...] = x1[:, :128]
    x1b[...] = x1[:, 128:]


def _node1(x, p1, A1, a1, A2, a2):
    full = lambda s: pl.BlockSpec(s, lambda i: (0,) * len(s))
    return pl.pallas_call(
        _node1_body,
        grid=(N // BN,),
        in_specs=[
            pl.BlockSpec((BN, 128), lambda i: (i, 0)),
            pl.BlockSpec((2, BN, 128), lambda i: (0, i, 0)),
            full((128, 256)), full((1, 256)),
            full((256, 256)), full((1, 256)),
        ],
        out_specs=[
            pl.BlockSpec((BN, 128), lambda i: (i, 0)),
            pl.BlockSpec((BN, 128), lambda i: (i, 0)),
        ],
        out_shape=[
            jax.ShapeDtypeStruct((N, 128), jnp.float32),
            jax.ShapeDtypeStruct((N, 128), jnp.float32),
        ],
    )(x, p1, A1, a1, A2, a2)


def _final_body(x1a, x1b, p2a, p2b, batch, B1, b1, B2, b2,
                L1, l1, L2, l2, H, hb, out, sums, cnt):
    i = pl.program_id(0)
    nsteps = pl.num_programs(0)

    @pl.when(i == 0)
    def _init():
        sums[...] = jnp.zeros_like(sums)
        cnt[...] = jnp.zeros_like(cnt)

    h = jnp.concatenate([x1a[...] + p2a[0] + p2a[1],
                         x1b[...] + p2b[0] + p2b[1]], axis=1)
    h = jnp.maximum(jnp.dot(h, B1[...], preferred_element_type=jnp.float32, precision=lax.Precision.HIGHEST) + b1[...], 0.0)
    h = jnp.dot(h, B2[...], preferred_element_type=jnp.float32, precision=lax.Precision.HIGHEST) + b2[...]
    x2 = jnp.maximum(h, 0.0)

    gid = lax.broadcasted_iota(jnp.int32, (BN, NGRAPH), 1)
    oh = jnp.where(batch[...] == gid, 1.0, 0.0).astype(jnp.float32)
    sums[...] += lax.dot_general(oh, x2, (((0,), (0,)), ((), ())),
                                 preferred_element_type=jnp.float32, precision=lax.Precision.HIGHEST)
    cnt[...] += lax.dot_general(oh, jnp.ones((BN, 1), jnp.float32),
                                (((0,), (0,)), ((), ())),
                                preferred_element_type=jnp.float32, precision=lax.Precision.HIGHEST)

    @pl.when(i == nsteps - 1)
    def _finish():
        g = sums[...] / jnp.maximum(cnt[...], 1.0)
        g = jnp.maximum(jnp.dot(g, L1[...], preferred_element_type=jnp.float32, precision=lax.Precision.HIGHEST) + l1[...], 0.0)
        g = jnp.maximum(jnp.dot(g, L2[...], preferred_element_type=jnp.float32, precision=lax.Precision.HIGHEST) + l2[...], 0.0)
        out[...] = jnp.dot(g, H[...], preferred_element_type=jnp.float32, precision=lax.Precision.HIGHEST) + hb[...]


def _final(x1a, x1b, p2a, p2b, batch2d, B1, b1, B2, b2, L1, l1, L2, l2, H, hb):
    full = lambda s: pl.BlockSpec(s, lambda i: (0,) * len(s))
    return pl.pallas_call(
        _final_body,
        grid=(N // BN,),
        in_specs=[
            pl.BlockSpec((BN, 128), lambda i: (i, 0)),
            pl.BlockSpec((BN, 128), lambda i: (i, 0)),
            pl.BlockSpec((2, BN, 128), lambda i: (0, i, 0)),
            pl.BlockSpec((2, BN, 128), lambda i: (0, i, 0)),
            pl.BlockSpec((BN, 1), lambda i: (i, 0)),
            full((256, 256)), full((1, 256)),
            full((256, 256)), full((1, 256)),
            full((256, 128)), full((1, 128)),
            full((128, 64)), full((1, 64)),
            full((64, 3)), full((1, 3)),
        ],
        out_specs=pl.BlockSpec((NGRAPH, 3), lambda i: (0, 0)),
        out_shape=jax.ShapeDtypeStruct((NGRAPH, 3), jnp.float32),
        scratch_shapes=[
            pltpu.VMEM((NGRAPH, 256), jnp.float32),
            pltpu.VMEM((NGRAPH, 1), jnp.float32),
        ],
    )(x1a, x1b, p2a, p2b, batch2d, B1, b1, B2, b2, L1, l1, L2, l2, H, hb)


def kernel(x, edge_index, edge_attr, batch, We1, be1, We2, be2, Wl1, bl1,
           A1, a1, A2, a2, Wl2, bl2, B1, b1, B2, b2, L1, l1, L2, l2,
           Hs, hs, Hp, hp, Hn, hn):
    src = edge_index[0].astype(jnp.int32)
    dst = edge_index[1].astype(jnp.int32)
    r = lambda b: b.reshape(1, -1)

    t1, t2a, t2b = _edge_mlp(edge_attr, We1, r(be1), We2, r(be2),
                             Wl1, r(bl1), Wl2, r(bl2))

    zeros = jnp.zeros((N_PAD, 128), jnp.float32)
    m1 = jnp.maximum(x[src] + t1, 0.0)
    p1 = _sc_scatter(m1, dst, zeros)
    x1a, x1b = _node1(x, p1, A1, r(a1), A2, r(a2))
    m2a = jnp.maximum(x1a[src] + t2a, 0.0)
    p2a = _sc_scatter(m2a, dst, zeros)
    x1b_dep = x1b + 0.0 * p2a[0, 0, 0]
    m2b = jnp.maximum(x1b_dep[src] + t2b, 0.0)
    p2b = _sc_scatter(m2b, dst, zeros)

    H = jnp.concatenate([Hs, Hp, Hn], axis=1)
    hb = jnp.stack([hs[0], hp[0], hn[0]]).reshape(1, 3)
    out = _final(x1a, x1b, p2a, p2b, batch.reshape(N, 1).astype(jnp.int32),
                 B1, r(b1), B2, r(b2), L1, r(l1), L2, r(l2), H, hb)
    return out[:, 0], out[:, 1], out[:, 2]
